# Initial kernel scaffold; baseline (speedup 1.0000x reference)
import jax
import jax.numpy as jnp
from jax.experimental import pallas as pl


def kernel(z, r, table):
    raise NotImplementedError("write your pallas kernel here")


# speedup vs baseline: 2.3648x; 2.3648x over previous
import functools

import jax
import jax.numpy as jnp
from jax import lax
from jax.experimental import pallas as pl
from jax.experimental.pallas import tpu as pltpu
from jax.experimental.pallas import tpu_sc as plsc

NC = 2
NS = 16
NW = NC * NS
LANES = 16
CHUNK = 128


def _make_lookup(n_rows, n_tab, d, dtype):
    per_w = n_rows // NW
    n_chunk = per_w // CHUNK
    mesh = plsc.VectorSubcoreMesh(core_axis_name="c", subcore_axis_name="s")

    @functools.partial(
        pl.kernel,
        out_type=jax.ShapeDtypeStruct((n_rows, d), dtype),
        mesh=mesh,
        scratch_types=[
            pltpu.VMEM((per_w,), jnp.int32),
            pltpu.VMEM((CHUNK, d), dtype),
            pltpu.VMEM((CHUNK, d), dtype),
            pltpu.SemaphoreType.DMA,
            pltpu.SemaphoreType.DMA,
            pltpu.SemaphoreType.DMA,
            pltpu.SemaphoreType.DMA,
        ],
    )
    def lookup(z_hbm, table_hbm, out_hbm, idx_v, rows0, rows1, g0, g1, p0, p1):
        wid = lax.axis_index("s") * NC + lax.axis_index("c")
        base = wid * per_w

        pltpu.sync_copy(z_hbm.at[pl.ds(base, per_w)], idx_v)

        def fix(i, carry):
            sl = pl.ds(i * LANES, LANES)
            v = idx_v[sl]
            idx_v[sl] = jnp.where(v > -1, v + 1, n_tab - 1)
            return carry

        lax.fori_loop(0, per_w // LANES, fix, 0)

        def gather(j, buf, sem):
            return pltpu.async_copy(
                table_hbm.at[idx_v.at[pl.ds(j * CHUNK, CHUNK)]], buf, sem)

        def put(j, buf, sem):
            return pltpu.async_copy(
                buf, out_hbm.at[pl.ds(base + j * CHUNK, CHUNK)], sem)

        def wait_put(buf, sem):
            pltpu.make_async_copy(
                buf, out_hbm.at[pl.ds(base, CHUNK)], sem).wait()

        def body(jj, carry):
            j0 = 2 * jj

            @pl.when(jj > 0)
            def _():
                wait_put(rows0, p0)

            ga = gather(j0, rows0, g0)

            @pl.when(jj > 0)
            def _():
                wait_put(rows1, p1)

            gb = gather(j0 + 1, rows1, g1)
            ga.wait()
            put(j0, rows0, p0)
            gb.wait()
            put(j0 + 1, rows1, p1)
            return carry

        lax.fori_loop(0, n_chunk // 2, body, 0)
        wait_put(rows0, p0)
        wait_put(rows1, p1)

    return lookup


def kernel(z, r, table):
    b, a = z.shape
    n_tab, d = table.shape
    zf = z.reshape(-1).astype(jnp.int32)
    tpad = jnp.concatenate([table, jnp.zeros((1, d), table.dtype)], axis=0)
    emb = _make_lookup(b * a, n_tab + 1, d, table.dtype)(zf, tpad)
    return (z, r, emb.reshape(b, a, d))
